# baseline (speedup 1.0000x reference)
import functools

import jax
import jax.numpy as jnp
from jax import lax
from jax.experimental import pallas as pl
from jax.experimental.pallas import tpu as pltpu
from jax.experimental.pallas import tpu_sc as plsc

SIZE = 1000000
DIM = 64
NB = 4096
T = 200
NC = 2
NS = 16
NW = NC * NS
N = NB * T
PER_W = N // NW
G = 128
BLOCKS = PER_W // G

_mesh = plsc.VectorSubcoreMesh(core_axis_name="c", subcore_axis_name="s")


@functools.partial(
    pl.kernel,
    mesh=_mesh,
    out_type=jax.ShapeDtypeStruct((N, DIM), jnp.float32),
    scratch_types=[
        pltpu.VMEM((PER_W,), jnp.int32),
        pltpu.VMEM((G, 2 * DIM), jnp.float32),
        pltpu.VMEM((G, DIM), jnp.float32),
        pltpu.SemaphoreType.DMA,
    ],
)
def _gather(x_hbm, t2_hbm, out_hbm, idx_v, rows_v, out_v, sem):
    wid = lax.axis_index("s") * NC + lax.axis_index("c")
    base = wid * PER_W
    pltpu.sync_copy(x_hbm.at[pl.ds(base, PER_W)], idx_v)

    def body(k, carry):
        pltpu.async_copy(
            t2_hbm.at[idx_v.at[pl.ds(k * G, G)]], rows_v, sem
        ).wait()

        def repack(g, c2):
            for u in range(4):
                for c0 in (0, 16, 32, 48):
                    out_v[g + u, pl.ds(c0, 16)] = rows_v[g + u, pl.ds(c0, 16)]
            return c2

        lax.fori_loop(0, G // 4, lambda g, c2: repack(4 * g, c2), 0, unroll=4)
        pltpu.sync_copy(out_v, out_hbm.at[pl.ds(base + k * G, G)])
        return carry

    lax.fori_loop(0, BLOCKS, body, 0)


def kernel(x, embed_weight):
    t2 = jnp.pad(embed_weight, ((0, 0), (0, DIM)))
    out = _gather(x.reshape(-1), t2)
    return out.reshape(NB, T, DIM)


# speedup vs baseline: 1.6187x; 1.6187x over previous
import functools

import jax
import jax.numpy as jnp
from jax import lax
from jax.experimental import pallas as pl
from jax.experimental.pallas import tpu as pltpu
from jax.experimental.pallas import tpu_sc as plsc

SIZE = 1000000
DIM = 64
NB = 4096
T = 200
NC = 2
NS = 16
NW = NC * NS
N = NB * T
PER_W = N // NW
G = 128
BLOCKS = PER_W // G

_mesh = plsc.VectorSubcoreMesh(core_axis_name="c", subcore_axis_name="s")


@functools.partial(
    pl.kernel,
    mesh=_mesh,
    out_type=jax.ShapeDtypeStruct((N, DIM), jnp.float32),
    scratch_types=[
        pltpu.VMEM((PER_W,), jnp.int32),
        pltpu.VMEM((2, G, 2 * DIM), jnp.float32),
        pltpu.VMEM((2, G, DIM), jnp.float32),
        pltpu.SemaphoreType.DMA((2,)),
        pltpu.SemaphoreType.DMA((2,)),
    ],
)
def _gather(x_hbm, t2_hbm, out_hbm, idx_v, rows_v, out_v, gsem, osem):
    wid = lax.axis_index("s") * NC + lax.axis_index("c")
    base = wid * PER_W
    pltpu.sync_copy(x_hbm.at[pl.ds(base, PER_W)], idx_v)

    def start_gather(k, slot):
        pltpu.async_copy(
            t2_hbm.at[idx_v.at[pl.ds(k * G, G)]], rows_v.at[slot], gsem.at[slot]
        )

    start_gather(0, 0)

    def body(k, carry):
        slot = lax.rem(k, 2)
        nslot = 1 - slot

        @pl.when(k + 1 < BLOCKS)
        def _():
            start_gather(k + 1, nslot)

        pltpu.make_async_copy(
            t2_hbm.at[idx_v.at[pl.ds(0, G)]], rows_v.at[slot], gsem.at[slot]
        ).wait()

        @pl.when(k >= 2)
        def _():
            pltpu.make_async_copy(
                out_v.at[slot], out_hbm.at[pl.ds(base, G)], osem.at[slot]
            ).wait()

        for g in range(G):
            for c0 in (0, 16, 32, 48):
                out_v[slot, g, pl.ds(c0, 16)] = rows_v[slot, g, pl.ds(c0, 16)]

        pltpu.async_copy(
            out_v.at[slot], out_hbm.at[pl.ds(base + k * G, G)], osem.at[slot]
        )
        return carry

    lax.fori_loop(0, BLOCKS, body, 0)

    pltpu.make_async_copy(
        out_v.at[0], out_hbm.at[pl.ds(base, G)], osem.at[0]
    ).wait()
    pltpu.make_async_copy(
        out_v.at[1], out_hbm.at[pl.ds(base, G)], osem.at[1]
    ).wait()


def kernel(x, embed_weight):
    t2 = jnp.pad(embed_weight, ((0, 0), (0, DIM)))
    out = _gather(x.reshape(-1), t2)
    return out.reshape(NB, T, DIM)
